# baseline (speedup 1.0000x reference)
import functools
import numpy as np
import jax
import jax.numpy as jnp
from jax import lax
from jax.experimental import pallas as pl
from jax.experimental.pallas import tpu as pltpu
from jax.experimental.pallas import tpu_sc as plsc

_N = 4194304

_E1 = np.complex64(np.exp(1j * np.pi / 4))
_PHI = np.float64(np.float32(np.pi)) / 4.0
_K = (_E1.astype(np.complex128)
      * np.exp(-1j * _PHI)
      * np.exp(-1j * np.pi / 4))
_K_RE = np.float32(_K.real)

_NC = 2
_NS = 16
_NW = _NC * _NS
_PER_W = _N // _NW
_CHUNK = 16384
_NCHUNK = _PER_W // _CHUNK
_UNROLL = 16
_VECS = _CHUNK // (16 * _UNROLL)

_mesh = plsc.VectorSubcoreMesh(core_axis_name="c", subcore_axis_name="s")


@functools.partial(
    pl.kernel,
    mesh=_mesh,
    out_type=jax.ShapeDtypeStruct((_N,), jnp.float32),
    scratch_types=[
        pltpu.VMEM((_CHUNK,), jnp.float32),
        pltpu.VMEM((_CHUNK,), jnp.float32),
        pltpu.VMEM((_CHUNK,), jnp.float32),
        pltpu.VMEM((_CHUNK,), jnp.float32),
        pltpu.SemaphoreType.DMA,
        pltpu.SemaphoreType.DMA,
        pltpu.SemaphoreType.DMA,
        pltpu.SemaphoreType.DMA,
    ],
)
def _sc_scale(x_hbm, out_hbm, in0, in1, ot0, ot1, si0, si1, so0, so1):
    wid = lax.axis_index("s") * _NC + lax.axis_index("c")
    base = wid * _PER_W
    ibufs, obufs = (in0, in1), (ot0, ot1)
    isems, osems = (si0, si1), (so0, so1)

    def start_in(g):
        off = base + g * _CHUNK
        return pltpu.async_copy(
            x_hbm.at[pl.ds(off, _CHUNK)], ibufs[g % 2], isems[g % 2])

    def start_out(g):
        off = base + g * _CHUNK
        return pltpu.async_copy(
            obufs[g % 2], out_hbm.at[pl.ds(off, _CHUNK)], osems[g % 2])

    hs_in = {0: start_in(0), 1: start_in(1)}
    hs_out = {}
    for g in range(_NCHUNK):
        b = g % 2
        hs_in[g].wait()
        if g >= 2:
            hs_out[g - 2].wait()
        src, dst = ibufs[b], obufs[b]

        def body(i, _):
            for u in range(_UNROLL):
                sl = pl.ds(i * (16 * _UNROLL) + u * 16, 16)
                dst[sl] = src[sl] * _K_RE
            return 0

        lax.fori_loop(0, _VECS, body, 0)
        hs_out[g] = start_out(g)
        if g + 2 < _NCHUNK:
            hs_in[g + 2] = start_in(g + 2)
    hs_out[_NCHUNK - 2].wait()
    hs_out[_NCHUNK - 1].wait()


def kernel(x):
    a = _sc_scale(x)
    return jax.lax.complex(a, -a)
